# baseline (speedup 1.0000x reference)
import functools

import jax
import jax.numpy as jnp
from jax.experimental import pallas as pl
from jax.experimental.pallas import tpu as pltpu

_BN_EPS = 1e-5


def _stats_kernel(x_ref, acc_ref, *, valid_rows, blocks_per_core, need_mask):
    c = pl.program_id(0)
    j = pl.program_id(1)

    @pl.when(j == 0)
    def _():
        acc_ref[...] = jnp.zeros_like(acc_ref)

    blk = x_ref.shape[0]
    x = x_ref[...]
    if need_mask:
        gb = c * blocks_per_core + j
        rows = gb * blk + jax.lax.broadcasted_iota(jnp.int32, x.shape, 0)
        x = jnp.where(rows < valid_rows, x, 0.0)

    ones = jnp.ones((8, blk), jnp.float32)
    acc_ref[0:8, :] += jnp.dot(ones, x, preferred_element_type=jnp.float32)
    acc_ref[8:16, :] += jnp.dot(ones, x * x, preferred_element_type=jnp.float32)


def _apply_kernel(acc_ref, w_ref, g_ref, bt_ref, bi_ref, x_ref, o_ref,
                  w_s, v_s, *, inv_n):
    j = pl.program_id(1)

    @pl.when(j == 0)
    def _():
        tot = jnp.sum(acc_ref[...], axis=0)
        sums = tot[0:1, :]
        sqs = tot[8:9, :]
        mean = sums * inv_n
        var = jnp.maximum(sqs * inv_n - mean * mean, 0.0)
        s = g_ref[...] * jax.lax.rsqrt(var + _BN_EPS)
        c0 = bt_ref[...] - mean * s
        w_s[...] = w_ref[...] * s
        v_s[0:1, :] = jax.lax.dot_general(
            c0, w_ref[...], (((1,), (1,)), ((), ())),
            preferred_element_type=jnp.float32) + bi_ref[...]

    xb = x_ref[...]
    y = jax.lax.dot_general(xb, w_s[...], (((1,), (1,)), ((), ())),
                            preferred_element_type=jnp.float32)
    o_ref[...] = (y + v_s[0:1, :]).astype(o_ref.dtype)


def _stats_pack_kernel(x0_ref, x1_ref, x2_ref, x3_ref, acc_ref, xp_ref):
    j = pl.program_id(1)

    @pl.when(j == 0)
    def _():
        acc_ref[...] = jnp.zeros_like(acc_ref)

    x4 = jnp.concatenate(
        [x0_ref[...], x1_ref[...], x2_ref[...], x3_ref[...]], axis=1)
    xp_ref[...] = x4
    blk = x4.shape[0]
    ones = jnp.ones((8, blk), jnp.float32)
    acc_ref[0:8, :] += jnp.dot(ones, x4, preferred_element_type=jnp.float32)
    acc_ref[8:16, :] += jnp.dot(ones, x4 * x4,
                                preferred_element_type=jnp.float32)


def _apply_unpack_kernel(acc_ref, wt_ref, g_ref, bt_ref, bi_ref, xp_ref,
                         o_ref, w_s, b_s, v_s, y_s, *, inv_n, d):
    j = pl.program_id(1)
    p = pl.program_id(2)
    pack = w_s.shape[0]

    @pl.when((j == 0) & (p == 0))
    def _():
        tot = jnp.sum(acc_ref[...], axis=0)
        sums = tot[0:1, :]
        sqs = tot[8:9, :]
        L = sums.shape[1]
        ii = jax.lax.broadcasted_iota(jnp.int32, (L, L), 0)
        jj = jax.lax.broadcasted_iota(jnp.int32, (L, L), 1)
        pm = ((ii % d) == (jj % d)).astype(jnp.float32)
        sp = jnp.dot(sums, pm, preferred_element_type=jnp.float32)
        qp = jnp.dot(sqs, pm, preferred_element_type=jnp.float32)
        mean = sp * inv_n
        var = jnp.maximum(qp * inv_n - mean * mean, 0.0)
        s = g_ref[...] * jax.lax.rsqrt(var + _BN_EPS)
        c0 = bt_ref[...] - mean * s
        wbd = jnp.where((ii // d) == (jj // d), wt_ref[...], 0.0)
        b2 = jnp.dot(c0, wbd, preferred_element_type=jnp.float32) \
            + bi_ref[...]
        v_s[0:1, :] = s
        for q in range(pack):
            w_s[q] = wbd[:, q * d:(q + 1) * d]
            b_s[q, 0:1, :] = b2[:, q * d:(q + 1) * d]

    @pl.when(p == 0)
    def _():
        xs = xp_ref[...] * v_s[0:1, :]
        for q in range(pack):
            y_s[q] = jnp.dot(xs, w_s[q],
                             preferred_element_type=jnp.float32) \
                + b_s[q, 0:1, :]

    o_ref[...] = y_s[p].astype(o_ref.dtype)


def _rwse_fast(x, gamma, beta, weight, bias, n, d, pack, blk):
    lanes = pack * d
    n4 = n // pack
    bpq = n4 // blk
    gh = bpq // 2

    def quarter_idx(q):
        return lambda c, j: (q * bpq + c * gh + j, 0)

    acc, x_packed = pl.pallas_call(
        _stats_pack_kernel,
        out_shape=(jax.ShapeDtypeStruct((2, 16, lanes), jnp.float32),
                   jax.ShapeDtypeStruct((n4, lanes), jnp.float32)),
        grid=(2, gh),
        in_specs=[pl.BlockSpec((blk, d), quarter_idx(q)) for q in range(pack)],
        out_specs=(pl.BlockSpec((None, 16, lanes), lambda c, j: (c, 0, 0)),
                   pl.BlockSpec((blk, lanes), lambda c, j: (c * gh + j, 0))),
        compiler_params=pltpu.CompilerParams(
            dimension_semantics=("parallel", "arbitrary"),
            vmem_limit_bytes=_VMEM_LIMIT),
    )(*[x for _ in range(pack)])

    out = pl.pallas_call(
        functools.partial(_apply_unpack_kernel, inv_n=1.0 / n, d=d),
        out_shape=jax.ShapeDtypeStruct((n, d), x.dtype),
        grid=(2, gh, pack),
        in_specs=[
            pl.BlockSpec((2, 16, lanes), lambda c, j, p: (0, 0, 0)),
            pl.BlockSpec((lanes, lanes), lambda c, j, p: (0, 0)),
            pl.BlockSpec((1, lanes), lambda c, j, p: (0, 0)),
            pl.BlockSpec((1, lanes), lambda c, j, p: (0, 0)),
            pl.BlockSpec((1, lanes), lambda c, j, p: (0, 0)),
            pl.BlockSpec((blk, lanes), lambda c, j, p: (c * gh + j, 0)),
        ],
        out_specs=pl.BlockSpec((blk, d), lambda c, j, p: (p * bpq + c * gh + j, 0)),
        scratch_shapes=[
            pltpu.VMEM((pack, lanes, d), jnp.float32),
            pltpu.VMEM((pack, 8, d), jnp.float32),
            pltpu.VMEM((8, lanes), jnp.float32),
            pltpu.VMEM((pack, blk, d), jnp.float32),
        ],
        compiler_params=pltpu.CompilerParams(
            dimension_semantics=("parallel", "arbitrary", "arbitrary"),
            vmem_limit_bytes=_VMEM_LIMIT),
    )(acc, jnp.tile(weight.T.astype(jnp.float32), (pack, pack)),
      jnp.tile(gamma.astype(jnp.float32), pack).reshape(1, lanes),
      jnp.tile(beta.astype(jnp.float32), pack).reshape(1, lanes),
      jnp.tile(bias.astype(jnp.float32), pack).reshape(1, lanes),
      x_packed)
    return out


def _split(n, block_rows):
    if n >= 8:
        blk = min(max(8, (int(block_rows) // 8) * 8), (n // 8) * 8)
    else:
        blk = n
    grid_n = pl.cdiv(n, blk)
    n_split = 2 if grid_n >= 2 else 1
    gh = pl.cdiv(grid_n, n_split)
    ragged = (gh * n_split != grid_n) or (grid_n * blk != n)

    def blk_idx(c, j):
        g = c * gh + j
        return ((jnp.minimum(g, grid_n - 1), 0) if ragged else (g, 0))

    return blk, grid_n, n_split, gh, ragged, blk_idx


_VMEM_LIMIT = 56 * 1024 * 1024


def kernel(x, gamma, beta, weight, bias, *,
           stats_block_rows=32768, block_rows=16384):
    n, d = x.shape

    pack = 128 // d if (d <= 128 and 128 % d == 0) else 1
    if pack == 4 and n % pack == 0:
        n4 = n // pack
        for blk in (8192, 4096, 2048, 1024, 512, 256, 128, 64, 32, 16, 8):
            if n4 % blk == 0 and (n4 // blk) % 2 == 0:
                return _rwse_fast(x, gamma, beta, weight, bias,
                                  n, d, pack, blk)

    sblk, _, s_split, sgh, sragged, sblk_idx = _split(n, stats_block_rows)
    acc = pl.pallas_call(
        functools.partial(_stats_kernel, valid_rows=n,
                          blocks_per_core=sgh, need_mask=sragged),
        out_shape=jax.ShapeDtypeStruct((s_split, 16, d), jnp.float32),
        grid=(s_split, sgh),
        in_specs=[pl.BlockSpec((sblk, d), sblk_idx)],
        out_specs=pl.BlockSpec((None, 16, d), lambda c, j: (c, 0, 0)),
        compiler_params=pltpu.CompilerParams(
            dimension_semantics=("parallel", "arbitrary"),
            vmem_limit_bytes=_VMEM_LIMIT),
    )(x)

    blk, grid_n, n_split, gh, ragged, blk_idx = _split(n, block_rows)

    out = pl.pallas_call(
        functools.partial(_apply_kernel, inv_n=1.0 / n),
        out_shape=jax.ShapeDtypeStruct((n, d), x.dtype),
        grid=(n_split, gh),
        in_specs=[
            pl.BlockSpec((s_split, 16, d), lambda c, j: (0, 0, 0)),
            pl.BlockSpec((d, d), lambda c, j: (0, 0)),
            pl.BlockSpec((1, d), lambda c, j: (0, 0)),
            pl.BlockSpec((1, d), lambda c, j: (0, 0)),
            pl.BlockSpec((1, d), lambda c, j: (0, 0)),
            pl.BlockSpec((blk, d), blk_idx),
        ],
        out_specs=pl.BlockSpec((blk, d), blk_idx),
        scratch_shapes=[
            pltpu.VMEM((d, d), jnp.float32),
            pltpu.VMEM((8, d), jnp.float32),
        ],
        compiler_params=pltpu.CompilerParams(
            dimension_semantics=("parallel", "arbitrary"),
            vmem_limit_bytes=_VMEM_LIMIT),
    )(acc, weight.astype(jnp.float32), gamma.reshape(1, d).astype(jnp.float32),
      beta.reshape(1, d).astype(jnp.float32), bias.reshape(1, d).astype(jnp.float32), x)

    return out


# speedup vs baseline: 1.2874x; 1.2874x over previous
import functools

import jax
import jax.numpy as jnp
from jax.experimental import pallas as pl
from jax.experimental.pallas import tpu as pltpu

_BN_EPS = 1e-5


def _stats_kernel(x_ref, acc_ref, *, valid_rows, blocks_per_core, need_mask):
    c = pl.program_id(0)
    j = pl.program_id(1)

    @pl.when(j == 0)
    def _():
        acc_ref[...] = jnp.zeros_like(acc_ref)

    blk = x_ref.shape[0]
    x = x_ref[...]
    if need_mask:
        gb = c * blocks_per_core + j
        rows = gb * blk + jax.lax.broadcasted_iota(jnp.int32, x.shape, 0)
        x = jnp.where(rows < valid_rows, x, 0.0)

    ones = jnp.ones((8, blk), jnp.float32)
    acc_ref[0:8, :] += jnp.dot(ones, x, preferred_element_type=jnp.float32)
    acc_ref[8:16, :] += jnp.dot(ones, x * x, preferred_element_type=jnp.float32)


def _apply_kernel(acc_ref, w_ref, g_ref, bt_ref, bi_ref, x_ref, o_ref,
                  w_s, v_s, *, inv_n):
    j = pl.program_id(1)

    @pl.when(j == 0)
    def _():
        tot = jnp.sum(acc_ref[...], axis=0)
        sums = tot[0:1, :]
        sqs = tot[8:9, :]
        mean = sums * inv_n
        var = jnp.maximum(sqs * inv_n - mean * mean, 0.0)
        s = g_ref[...] * jax.lax.rsqrt(var + _BN_EPS)
        c0 = bt_ref[...] - mean * s
        w_s[...] = w_ref[...] * s
        v_s[0:1, :] = jax.lax.dot_general(
            c0, w_ref[...], (((1,), (1,)), ((), ())),
            preferred_element_type=jnp.float32) + bi_ref[...]

    xb = x_ref[...]
    y = jax.lax.dot_general(xb, w_s[...], (((1,), (1,)), ((), ())),
                            preferred_element_type=jnp.float32)
    o_ref[...] = (y + v_s[0:1, :]).astype(o_ref.dtype)


def _stats_pack_kernel(x0_ref, x1_ref, x2_ref, x3_ref, acc_ref, xp_ref):
    j = pl.program_id(1)

    @pl.when(j == 0)
    def _():
        acc_ref[...] = jnp.zeros_like(acc_ref)

    blk = xp_ref.shape[0]
    x4 = jnp.concatenate(
        [x0_ref[...], x1_ref[...], x2_ref[...], x3_ref[...]], axis=1)
    xp_ref[...] = x4
    ones = jnp.ones((8, blk), jnp.float32)
    acc_ref[0:8, :] += jnp.dot(ones, x4, preferred_element_type=jnp.float32)
    acc_ref[8:16, :] += jnp.dot(ones, x4 * x4,
                                preferred_element_type=jnp.float32)


def _apply_packed_kernel(acc_ref, wt_ref, g_ref, bt_ref, bi_ref, xp_ref,
                         o_ref, w_s, v_s, *, inv_n, d):
    j = pl.program_id(1)

    @pl.when(j == 0)
    def _():
        tot = jnp.sum(acc_ref[...], axis=0)
        sums = tot[0:1, :]
        sqs = tot[8:9, :]
        L = sums.shape[1]
        ii = jax.lax.broadcasted_iota(jnp.int32, (L, L), 0)
        jj = jax.lax.broadcasted_iota(jnp.int32, (L, L), 1)
        pm = ((ii % d) == (jj % d)).astype(jnp.float32)
        sp = jnp.dot(sums, pm, preferred_element_type=jnp.float32)
        qp = jnp.dot(sqs, pm, preferred_element_type=jnp.float32)
        mean = sp * inv_n
        var = jnp.maximum(qp * inv_n - mean * mean, 0.0)
        s = g_ref[...] * jax.lax.rsqrt(var + _BN_EPS)
        c0 = bt_ref[...] - mean * s
        wbd = jnp.where((ii // d) == (jj // d), wt_ref[...], 0.0)
        w_s[...] = wbd
        v_s[0:1, :] = s
        v_s[1:2, :] = jnp.dot(c0, wbd, preferred_element_type=jnp.float32) \
            + bi_ref[...]

    xs = xp_ref[...] * v_s[0:1, :]
    y = jnp.dot(xs, w_s[...], preferred_element_type=jnp.float32)
    o_ref[...] = (y + v_s[1:2, :]).astype(o_ref.dtype)


def _rwse_fast(x, gamma, beta, weight, bias, n, d, pack, blk):
    lanes = pack * d
    n4 = n // pack
    bpq = n4 // blk
    gh = bpq // 2

    def quarter_idx(q):
        return lambda c, j: (q * bpq + c * gh + j, 0)

    acc, x_packed = pl.pallas_call(
        _stats_pack_kernel,
        out_shape=(jax.ShapeDtypeStruct((2, 16, lanes), jnp.float32),
                   jax.ShapeDtypeStruct((n4, lanes), jnp.float32)),
        grid=(2, gh),
        in_specs=[pl.BlockSpec((blk, d), quarter_idx(q)) for q in range(pack)],
        out_specs=(pl.BlockSpec((None, 16, lanes), lambda c, j: (c, 0, 0)),
                   pl.BlockSpec((blk, lanes), lambda c, j: (c * gh + j, 0))),
        compiler_params=pltpu.CompilerParams(
            dimension_semantics=("parallel", "arbitrary"),
            vmem_limit_bytes=_VMEM_LIMIT),
    )(*[x for _ in range(pack)])

    out_packed = pl.pallas_call(
        functools.partial(_apply_packed_kernel, inv_n=1.0 / n, d=d),
        out_shape=jax.ShapeDtypeStruct((n4, lanes), x.dtype),
        grid=(2, gh),
        in_specs=[
            pl.BlockSpec((2, 16, lanes), lambda c, j: (0, 0, 0)),
            pl.BlockSpec((lanes, lanes), lambda c, j: (0, 0)),
            pl.BlockSpec((1, lanes), lambda c, j: (0, 0)),
            pl.BlockSpec((1, lanes), lambda c, j: (0, 0)),
            pl.BlockSpec((1, lanes), lambda c, j: (0, 0)),
            pl.BlockSpec((blk, lanes), lambda c, j: (c * gh + j, 0)),
        ],
        out_specs=pl.BlockSpec((blk, lanes), lambda c, j: (c * gh + j, 0)),
        scratch_shapes=[
            pltpu.VMEM((lanes, lanes), jnp.float32),
            pltpu.VMEM((8, lanes), jnp.float32),
        ],
        compiler_params=pltpu.CompilerParams(
            dimension_semantics=("parallel", "arbitrary"),
            vmem_limit_bytes=_VMEM_LIMIT),
    )(acc, jnp.tile(weight.T.astype(jnp.float32), (pack, pack)),
      jnp.tile(gamma.astype(jnp.float32), pack).reshape(1, lanes),
      jnp.tile(beta.astype(jnp.float32), pack).reshape(1, lanes),
      jnp.tile(bias.astype(jnp.float32), pack).reshape(1, lanes),
      x_packed)
    return out_packed.reshape(n4, pack, d).transpose(1, 0, 2).reshape(n, d)


def _split(n, block_rows):
    if n >= 8:
        blk = min(max(8, (int(block_rows) // 8) * 8), (n // 8) * 8)
    else:
        blk = n
    grid_n = pl.cdiv(n, blk)
    n_split = 2 if grid_n >= 2 else 1
    gh = pl.cdiv(grid_n, n_split)
    ragged = (gh * n_split != grid_n) or (grid_n * blk != n)

    def blk_idx(c, j):
        g = c * gh + j
        return ((jnp.minimum(g, grid_n - 1), 0) if ragged else (g, 0))

    return blk, grid_n, n_split, gh, ragged, blk_idx


_VMEM_LIMIT = 56 * 1024 * 1024


def kernel(x, gamma, beta, weight, bias, *,
           stats_block_rows=32768, block_rows=16384):
    n, d = x.shape

    pack = 128 // d if (d <= 128 and 128 % d == 0) else 1
    if pack == 4 and n % pack == 0:
        n4 = n // pack
        for blk in (8192, 4096, 2048, 1024, 512, 256, 128, 64, 32, 16, 8):
            if n4 % blk == 0 and (n4 // blk) % 2 == 0:
                return _rwse_fast(x, gamma, beta, weight, bias,
                                  n, d, pack, blk)

    sblk, _, s_split, sgh, sragged, sblk_idx = _split(n, stats_block_rows)
    acc = pl.pallas_call(
        functools.partial(_stats_kernel, valid_rows=n,
                          blocks_per_core=sgh, need_mask=sragged),
        out_shape=jax.ShapeDtypeStruct((s_split, 16, d), jnp.float32),
        grid=(s_split, sgh),
        in_specs=[pl.BlockSpec((sblk, d), sblk_idx)],
        out_specs=pl.BlockSpec((None, 16, d), lambda c, j: (c, 0, 0)),
        compiler_params=pltpu.CompilerParams(
            dimension_semantics=("parallel", "arbitrary"),
            vmem_limit_bytes=_VMEM_LIMIT),
    )(x)

    blk, grid_n, n_split, gh, ragged, blk_idx = _split(n, block_rows)

    out = pl.pallas_call(
        functools.partial(_apply_kernel, inv_n=1.0 / n),
        out_shape=jax.ShapeDtypeStruct((n, d), x.dtype),
        grid=(n_split, gh),
        in_specs=[
            pl.BlockSpec((s_split, 16, d), lambda c, j: (0, 0, 0)),
            pl.BlockSpec((d, d), lambda c, j: (0, 0)),
            pl.BlockSpec((1, d), lambda c, j: (0, 0)),
            pl.BlockSpec((1, d), lambda c, j: (0, 0)),
            pl.BlockSpec((1, d), lambda c, j: (0, 0)),
            pl.BlockSpec((blk, d), blk_idx),
        ],
        out_specs=pl.BlockSpec((blk, d), blk_idx),
        scratch_shapes=[
            pltpu.VMEM((d, d), jnp.float32),
            pltpu.VMEM((8, d), jnp.float32),
        ],
        compiler_params=pltpu.CompilerParams(
            dimension_semantics=("parallel", "arbitrary"),
            vmem_limit_bytes=_VMEM_LIMIT),
    )(acc, weight.astype(jnp.float32), gamma.reshape(1, d).astype(jnp.float32),
      beta.reshape(1, d).astype(jnp.float32), bias.reshape(1, d).astype(jnp.float32), x)

    return out
